# baseline (speedup 1.0000x reference)
import functools

import jax
import jax.numpy as jnp
import numpy as np
from jax import lax
from jax.experimental import pallas as pl
from jax.experimental.pallas import tpu as pltpu
from jax.experimental.pallas import tpu_sc as plsc

_VERT_IDX = np.array([1382, 1522, 1541, 1667, 1493, 428, 179, 1806, 2289,
                      2408, 2405, 2442, 19, 2504, 3016, 3164, 3049, 3060,
                      364, 626, 3454, 3756, 3863, 3844, 3915, 0, 0, 0, 0,
                      0, 0, 0, 2420, 2332, 2131, 2241, 3129, 3133, 2895,
                      3005, 3815, 3778, 3644, 3713, 0, 0], dtype=np.int64)

_B = 4096
_V = 4040
_K = _VERT_IDX.shape[0]

_NW = 32
_BPW = _B // _NW
_RPW = _BPW * _K
_NCHUNK = _RPW // 128

_FLAT_IDX = (np.arange(_B, dtype=np.int64)[:, None] * _V
             + _VERT_IDX[None, :]).astype(np.int32).reshape(
                 _NW, _NCHUNK, 128)


def _gather_kernel(table_hbm, idx_hbm, out_hbm, idx_v, rows_v, sem):
    wid = lax.axis_index("s") * 2 + lax.axis_index("c")
    pltpu.sync_copy(idx_hbm.at[pl.ds(wid * _NCHUNK, _NCHUNK)], idx_v)

    def body(j, carry):
        pltpu.async_copy(
            table_hbm.at[idx_v.at[j]],
            rows_v.at[pl.ds(j * 128, 128)],
            sem,
        ).wait()
        return carry

    lax.fori_loop(0, _NCHUNK, body, 0)
    pltpu.sync_copy(rows_v, out_hbm.at[pl.ds(wid * _RPW, _RPW)])


def kernel(vertices):
    table = vertices.reshape(_B * _V, 3)
    idx = jnp.asarray(_FLAT_IDX.reshape(_NW * _NCHUNK, 128))
    mesh = plsc.VectorSubcoreMesh(core_axis_name="c", subcore_axis_name="s")
    run = functools.partial(
        pl.kernel,
        mesh=mesh,
        out_type=jax.ShapeDtypeStruct((_B * _K, 3), jnp.float32),
        scratch_types=[
            pltpu.VMEM((_NCHUNK, 128), jnp.int32),
            pltpu.VMEM((_RPW, 3), jnp.float32),
            pltpu.SemaphoreType.DMA,
        ],
        compiler_params=pltpu.CompilerParams(use_tc_tiling_on_sc=False),
    )(_gather_kernel)
    out = run(table, idx)
    return out.reshape(_B, _K, 3)


# speedup vs baseline: 1.0790x; 1.0790x over previous
import functools

import jax
import jax.numpy as jnp
import numpy as np
from jax import lax
from jax.experimental import pallas as pl
from jax.experimental.pallas import tpu as pltpu
from jax.experimental.pallas import tpu_sc as plsc

_VERT_IDX = np.array([1382, 1522, 1541, 1667, 1493, 428, 179, 1806, 2289,
                      2408, 2405, 2442, 19, 2504, 3016, 3164, 3049, 3060,
                      364, 626, 3454, 3756, 3863, 3844, 3915, 0, 0, 0, 0,
                      0, 0, 0, 2420, 2332, 2131, 2241, 3129, 3133, 2895,
                      3005, 3815, 3778, 3644, 3713, 0, 0], dtype=np.int64)

_B = 4096
_V = 4040
_K = _VERT_IDX.shape[0]
_ROW_STRIDE = _V * 3

_UNIQ, _INV = np.unique(_VERT_IDX, return_inverse=True)
_NU = _UNIQ.shape[0]

_NW = 32
_BPW = _B // _NW
_GPW = _BPW * _NU * 2
_NCHUNK = _GPW // 128
_OPW = _BPW * _K * 3

_o = (np.arange(_B, dtype=np.int64)[:, None] * _ROW_STRIDE
      + 3 * _UNIQ[None, :])
_r0 = _o // 8
_GIDX = np.stack([_r0, _r0 + 1], axis=-1).astype(np.int32).reshape(
    _NW, _NCHUNK, 128)

_p = np.arange(8 * _K * 3, dtype=np.int64)
_bl2, _j = _p // (_K * 3), _p % (_K * 3)
_k, _c = _j // 3, _j % 3
_q = (3 * _VERT_IDX[_k]) % 8
_s = _q + _c
_PAT_ROW = (2 * (_bl2 * _NU + _INV[_k]) + _s // 8).astype(np.int32)
_PAT_COL = (_s % 8).astype(np.int32)
_SB_ROWS = 2 * 8 * _NU
_NSB = _BPW // 8
_NVEC = (8 * _K * 3) // 16


def _anchor_kernel(table_hbm, gidx_hbm, patr_hbm, patc_hbm, out_hbm,
                   gidx_v, rows_v, patr_v, patc_v, outbuf_v, sem):
    wid = lax.axis_index("s") * 2 + lax.axis_index("c")
    pltpu.sync_copy(gidx_hbm.at[pl.ds(wid * _NCHUNK, _NCHUNK)], gidx_v)
    pltpu.sync_copy(patr_hbm, patr_v)
    pltpu.sync_copy(patc_hbm, patc_v)

    def fire(j, carry):
        pltpu.async_copy(
            table_hbm.at[gidx_v.at[j]],
            rows_v.at[pl.ds(j * 128, 128)],
            sem,
        )
        return carry

    lax.fori_loop(0, _NCHUNK, fire, 0)
    pltpu.make_async_copy(table_hbm.at[pl.ds(0, _GPW)], rows_v, sem).wait()

    def extract(t, carry):
        row_base = t * _SB_ROWS
        out_base = t * (_NVEC * 16)
        for i in range(_NVEC):
            rows = patr_v[pl.ds(i * 16, 16)] + row_base
            cols = patc_v[pl.ds(i * 16, 16)]
            vals = plsc.load_gather(rows_v, [rows, cols])
            outbuf_v[pl.ds(out_base + i * 16, 16)] = vals
        return carry

    lax.fori_loop(0, _NSB, extract, 0)
    pltpu.sync_copy(outbuf_v, out_hbm.at[pl.ds(wid * _OPW, _OPW)])


def kernel(vertices):
    table = vertices.reshape(_B * _V * 3 // 8, 8)
    gidx = jnp.asarray(_GIDX.reshape(_NW * _NCHUNK, 128))
    patr = jnp.asarray(_PAT_ROW)
    patc = jnp.asarray(_PAT_COL)
    mesh = plsc.VectorSubcoreMesh(core_axis_name="c", subcore_axis_name="s")
    run = functools.partial(
        pl.kernel,
        mesh=mesh,
        out_type=jax.ShapeDtypeStruct((_B * _K * 3,), jnp.float32),
        scratch_types=[
            pltpu.VMEM((_NCHUNK, 128), jnp.int32),
            pltpu.VMEM((_GPW, 8), jnp.float32),
            pltpu.VMEM((8 * _K * 3,), jnp.int32),
            pltpu.VMEM((8 * _K * 3,), jnp.int32),
            pltpu.VMEM((_OPW,), jnp.float32),
            pltpu.SemaphoreType.DMA,
        ],
        compiler_params=pltpu.CompilerParams(
            use_tc_tiling_on_sc=False, needs_layout_passes=False),
    )(_anchor_kernel)
    out = run(table, gidx, patr, patc)
    return out.reshape(_B, _K, 3)


# speedup vs baseline: 14.0540x; 13.0248x over previous
import jax
import jax.numpy as jnp
import numpy as np
from jax.experimental import pallas as pl
from jax.experimental.pallas import tpu as pltpu

_VERT_IDX = np.array([1382, 1522, 1541, 1667, 1493, 428, 179, 1806, 2289,
                      2408, 2405, 2442, 19, 2504, 3016, 3164, 3049, 3060,
                      364, 626, 3454, 3756, 3863, 3844, 3915, 0, 0, 0, 0,
                      0, 0, 0, 2420, 2332, 2131, 2241, 3129, 3133, 2895,
                      3005, 3815, 3778, 3644, 3713, 0, 0], dtype=np.int64)

_B = 4096
_V = 4040
_K = _VERT_IDX.shape[0]
_TB = 512
_GRID = _B // _TB


def _gather_body(vert_ref, out_ref, sem):
    i = pl.program_id(0)
    b0 = i * _TB

    def copy(k):
        return pltpu.make_async_copy(
            vert_ref.at[pl.ds(b0, _TB), pl.ds(int(_VERT_IDX[k]), 1), :],
            out_ref.at[:, pl.ds(k, 1), :],
            sem.at[k],
        )

    for k in range(_K):
        copy(k).start()
    for k in range(_K):
        copy(k).wait()


def kernel(vertices):
    return pl.pallas_call(
        _gather_body,
        grid=(_GRID,),
        in_specs=[pl.BlockSpec(memory_space=pltpu.MemorySpace.HBM)],
        out_specs=pl.BlockSpec((_TB, _K, 3), lambda i: (i, 0, 0)),
        out_shape=jax.ShapeDtypeStruct((_B, _K, 3), jnp.float32),
        scratch_shapes=[pltpu.SemaphoreType.DMA((_K,))],
    )(vertices)


# speedup vs baseline: 16988.3026x; 1208.7865x over previous
import jax
import jax.numpy as jnp
import numpy as np
from jax.experimental import pallas as pl
from jax.experimental.pallas import tpu as pltpu

_VERT_IDX = np.array([1382, 1522, 1541, 1667, 1493, 428, 179, 1806, 2289,
                      2408, 2405, 2442, 19, 2504, 3016, 3164, 3049, 3060,
                      364, 626, 3454, 3756, 3863, 3844, 3915, 0, 0, 0, 0,
                      0, 0, 0, 2420, 2332, 2131, 2241, 3129, 3133, 2895,
                      3005, 3815, 3778, 3644, 3713, 0, 0], dtype=np.int64)

_B = 4096
_V = 4040
_K = _VERT_IDX.shape[0]


def _gather_body(vt_ref, out_ref, sem):
    def copy(c, k):
        return pltpu.make_async_copy(
            vt_ref.at[pl.ds(c, 1), pl.ds(int(_VERT_IDX[k]), 1), :],
            out_ref.at[pl.ds(c, 1), pl.ds(k, 1), :],
            sem,
        )

    for c in range(3):
        for k in range(_K):
            copy(c, k).start()
    for c in range(3):
        for k in range(_K):
            copy(c, k).wait()


def kernel(vertices):
    vt = jnp.transpose(vertices, (2, 1, 0))
    out_t = pl.pallas_call(
        _gather_body,
        in_specs=[pl.BlockSpec(memory_space=pltpu.MemorySpace.HBM)],
        out_specs=pl.BlockSpec(memory_space=pltpu.MemorySpace.VMEM),
        out_shape=jax.ShapeDtypeStruct((3, _K, _B), jnp.float32),
        scratch_shapes=[pltpu.SemaphoreType.DMA],
    )(vt)
    return jnp.transpose(out_t, (2, 1, 0))
